# baseline (speedup 1.0000x reference)
import functools

import jax
import jax.numpy as jnp
from jax import lax
from jax.experimental import pallas as pl
from jax.experimental.pallas import tpu as pltpu
from jax.experimental.pallas import tpu_sc as plsc

R_REAL_C = 0.5
R_FAKE_C = 0.2
ALPHA_C = 20.0

N = 16384
L = 16
NT = 16
PER_T = N // NT
CHUNKS = PER_T // L


def _softplus_vec(v):
    a = jnp.abs(v)
    u = jnp.exp(-a)
    z = u / (2.0 + u)
    z2 = z * z
    l1p = 2.0 * z * (1.0 + z2 * (1.0 / 3.0 + z2 * (0.2 + z2 * (1.0 / 7.0 + z2 * (1.0 / 9.0)))))
    return jnp.maximum(v, 0.0) + l1p


_mesh = plsc.VectorSubcoreMesh(
    core_axis_name="c", subcore_axis_name="s", num_cores=1
)


@functools.partial(
    pl.kernel,
    out_type=jax.ShapeDtypeStruct((L,), jnp.float32),
    mesh=_mesh,
    scratch_types=[
        pltpu.VMEM((PER_T,), jnp.float32),
        pltpu.VMEM((PER_T,), jnp.int32),
        pltpu.VMEM((L,), jnp.float32),
        pltpu.VMEM_SHARED((NT * L,), jnp.float32),
        pltpu.VMEM((NT * L,), jnp.float32),
    ],
)
def _oc_loss_kernel(x_hbm, lab_hbm, out_hbm, xv, lv, pv, shared, gbuf):
    sid = lax.axis_index("s")
    base = sid * PER_T
    pltpu.sync_copy(x_hbm.at[pl.ds(base, PER_T)], xv)
    pltpu.sync_copy(lab_hbm.at[pl.ds(base, PER_T)], lv)

    def body(i, acc):
        xs = xv[pl.ds(i * L, L)]
        lb = lv[pl.ds(i * L, L)]
        out = jnp.where(lb == 0, R_REAL_C - xs,
                        jnp.where(lb == 1, xs - R_FAKE_C, xs))
        return acc + _softplus_vec(ALPHA_C * out)

    acc = lax.fori_loop(0, CHUNKS, body, jnp.zeros((L,), jnp.float32))
    pv[...] = acc
    pltpu.sync_copy(pv, shared.at[pl.ds(sid * L, L)])
    plsc.subcore_barrier()

    @pl.when(sid == 0)
    def _():
        pltpu.sync_copy(shared, gbuf)
        tot = gbuf[pl.ds(0, L)]
        for r in range(1, NT):
            tot = tot + gbuf[pl.ds(r * L, L)]
        total = tot[0]
        for i in range(1, L):
            total = total + tot[i]
        pv[...] = jnp.broadcast_to(total * (1.0 / N), (L,))
        pltpu.sync_copy(pv, out_hbm)


def kernel(x, labels):
    xf = jnp.reshape(x, (N,))
    out = _oc_loss_kernel(xf, labels)
    return out[0]


# speedup vs baseline: 11.7320x; 11.7320x over previous
import functools

import jax
import jax.numpy as jnp
from jax.experimental import pallas as pl
from jax.experimental.pallas import tpu as pltpu

R_REAL_C = 0.5
R_FAKE_C = 0.2
ALPHA_C = 20.0

N = 16384
ROWS = 128
COLS = 128


def _loss_body(x_ref, lab_ref, o_ref):
    xs = x_ref[...]
    lb = lab_ref[...]
    out = jnp.where(lb == 0, R_REAL_C - xs,
                    jnp.where(lb == 1, xs - R_FAKE_C, xs))
    v = ALPHA_C * out
    sp = jnp.maximum(v, 0.0) + jnp.log1p(jnp.exp(-jnp.abs(v)))
    o_ref[0, 0] = jnp.sum(sp) * (1.0 / N)


_loss_call = pl.pallas_call(
    _loss_body,
    out_shape=jax.ShapeDtypeStruct((1, 1), jnp.float32),
    in_specs=[
        pl.BlockSpec(memory_space=pltpu.VMEM),
        pl.BlockSpec(memory_space=pltpu.VMEM),
    ],
    out_specs=pl.BlockSpec(memory_space=pltpu.SMEM),
)


def kernel(x, labels):
    xf = jnp.reshape(x, (ROWS, COLS))
    lf = jnp.reshape(labels, (ROWS, COLS))
    return _loss_call(xf, lf)[0, 0]
